# baseline (speedup 1.0000x reference)
import functools

import jax
import jax.numpy as jnp
from jax import lax
from jax.experimental import pallas as pl
from jax.experimental.pallas import tpu as pltpu
from jax.experimental.pallas import tpu_sc as plsc

_NS = 8
_L = 16


def _dense_body(x_ref, row_ref, cs_ref):
    j = pl.program_id(1)
    xb = x_ref[0]
    G, _, W = xb.shape
    colmax = jnp.max(xb, axis=1)
    gm = jnp.max(colmax, axis=1, keepdims=True)
    ties = (xb >= gm[:, :, None]).astype(jnp.float32)
    ones = jnp.ones((G, 1, W), jnp.float32)
    rp = jax.lax.dot_general(
        ones, ties, (((2,), (2,)), ((0,), (0,))),
        preferred_element_type=jnp.float32)
    partial = jnp.sum(rp[:, 0, :], axis=0)

    @pl.when(j == 0)
    def _():
        row_ref[0, 0, :] = partial

    @pl.when(j > 0)
    def _():
        row_ref[0, 0, :] = row_ref[0, 0, :] + partial

    cs_ref[0, 0, 0, :] = jnp.sum(jnp.sum(xb, axis=1), axis=1)


def _post_body(row_ref, cs_ref, out_ref, *, C, H, W):
    row = row_ref[:, 0, :]
    B = row.shape[0]
    tri = (jax.lax.broadcasted_iota(jnp.int32, (H, H), 0)
           < jax.lax.broadcasted_iota(jnp.int32, (H, H), 1)).astype(jnp.float32)
    Hh = jax.lax.dot_general(row, tri, (((1,), (0,)), ((), ())),
                             preferred_element_type=jnp.float32)
    lane = jax.lax.broadcasted_iota(jnp.int32, (B, H), 1).astype(jnp.float32)
    jlo, jhi = 1.0, float(H - 2)
    inrange = (lane >= jlo) & (lane <= jhi)
    Hnext = jnp.concatenate([Hh[:, 1:], jnp.zeros((B, 1), jnp.float32)], axis=1)
    BIG = jnp.float32(1e9)
    hk_prev = jnp.zeros((B, 1), jnp.float32)
    valid = jnp.ones((B, 1), jnp.bool_)
    hks = [jnp.zeros((B, 1), jnp.float32)]
    for k in range(1, _NS):
        thr = float(int(k * C / _NS))
        cond = inrange & (Hh <= thr) & (Hnext > thr)
        a = jnp.min(jnp.where(cond, lane, BIG), axis=1, keepdims=True)
        b = jnp.max(jnp.where(cond, lane, -BIG), axis=1, keepdims=True)
        jk = jnp.maximum(a, hk_prev + 1.0)
        valid = valid & (jk <= b)
        hks.append(jnp.where(valid, jk, 0.0))
        hk_prev = jnp.where(valid, jk, hk_prev)
    hks.append(jnp.full((B, 1), jnp.float32(H)))
    hks = jnp.concatenate(hks, axis=1)
    widths = hks[:, 1:] - hks[:, :-1]
    F = cs_ref[:, 0, :] * jnp.float32(1.0 / W)
    out_ref[...] = F[:, None, :] / widths[:, :, None]


def _sc_post(row2, cs2, *, B, C, H, W):
    mesh = plsc.VectorSubcoreMesh(core_axis_name="c", subcore_axis_name="s")
    nch = H // _L
    ncc = C // _L
    thrs = [float(int(k * C / _NS)) for k in range(1, _NS)]

    @functools.partial(
        pl.kernel, mesh=mesh,
        out_type=jax.ShapeDtypeStruct((B, _NS * C), jnp.float32),
        scratch_types=[
            pltpu.VMEM((H,), jnp.float32),
            pltpu.VMEM((C,), jnp.float32),
            pltpu.VMEM((_NS * C,), jnp.float32),
        ],
    )
    def k(row_hbm, cs_hbm, out_hbm, row_v, cs_v, out_v):
        wid = lax.axis_index("s") * 2 + lax.axis_index("c")
        io = lax.iota(jnp.int32, _L)
        last = jnp.full((_L,), _L - 1, jnp.int32)

        @pl.when(wid < B)
        def _():
            b = wid
            pltpu.sync_copy(row_hbm.at[b], row_v)
            pltpu.sync_copy(cs_hbm.at[b], cs_v)
            big = jnp.full((_L,), 1e9, jnp.float32)
            zero = jnp.zeros((_L,), jnp.float32)
            carry = zero
            amins = [big] * (_NS - 1)
            bmaxs = [-big] * (_NS - 1)
            for ci in range(nch):
                chunk = row_v[pl.ds(ci * _L, _L)]
                inc = chunk
                for d in (1, 2, 4, 8):
                    g = inc[jnp.maximum(io - d, 0)]
                    inc = inc + jnp.where(io >= d, g, zero)
                hnext = carry + inc
                hcur = hnext - chunk
                carry = carry + inc[last]
                jv = io.astype(jnp.float32) + float(ci * _L)
                ok = (jv >= 1.0) & (jv <= float(H - 2))
                for t in range(_NS - 1):
                    cond = ok & (hcur <= thrs[t]) & (hnext > thrs[t])
                    amins[t] = jnp.minimum(amins[t], jnp.where(cond, jv, big))
                    bmaxs[t] = jnp.maximum(bmaxs[t], jnp.where(cond, jv, -big))
            hks = [zero]
            prev = zero
            valid = (io == io)
            one = jnp.ones((_L,), jnp.float32)
            for t in range(_NS - 1):
                a, bm = amins[t], bmaxs[t]
                for d in (1, 2, 4, 8):
                    a = jnp.minimum(a, a[io ^ d])
                    bm = jnp.maximum(bm, bm[io ^ d])
                jk = jnp.maximum(a, prev + one)
                valid = valid & (jk <= bm)
                hk = jnp.where(valid, jk, zero)
                hks.append(hk)
                prev = jnp.where(valid, jk, prev)
            hks.append(jnp.full((_L,), float(H), jnp.float32))
            inv_w = jnp.float32(1.0 / W)
            for s in range(_NS):
                wdt = hks[s + 1] - hks[s]
                for cc in range(ncc):
                    f = cs_v[pl.ds(cc * _L, _L)] * inv_w
                    out_v[pl.ds(s * C + cc * _L, _L)] = f / wdt
            pltpu.sync_copy(out_v, out_hbm.at[b])

    return k(row2, cs2)


def _dense(x, b0, nb):
    B, C, H, W = x.shape
    G = 96
    while C % G:
        G -= 1
    nj = C // G
    row, cs = pl.pallas_call(
        _dense_body,
        grid=(nb, nj),
        in_specs=[pl.BlockSpec((1, G, H, W), lambda b, j: (b + b0, j, 0, 0))],
        out_specs=[
            pl.BlockSpec((1, 1, H), lambda b, j: (b, 0, 0)),
            pl.BlockSpec((1, 1, 1, G), lambda b, j: (b, j, 0, 0)),
        ],
        out_shape=[
            jax.ShapeDtypeStruct((nb, 1, H), jnp.float32),
            jax.ShapeDtypeStruct((nb, nj, 1, G), jnp.float32),
        ],
        compiler_params=pltpu.CompilerParams(
            dimension_semantics=("parallel", "arbitrary")),
    )(x)
    return row.reshape(nb, H), cs.reshape(nb, C)


def _abp(x):
    B, C, H, W = x.shape
    nbA = B // 2
    rowA, csA = _dense(x, 0, nbA)
    outA = _sc_post(rowA, csA, B=nbA, C=C, H=H, W=W)
    rowB, csB = _dense(x, nbA, B - nbA)
    outB = _sc_post(rowB, csB, B=B - nbA, C=C, H=H, W=W)
    return jnp.concatenate([outA, outB], axis=0)


def kernel(x):
    return _abp(x)


# speedup vs baseline: 1.4174x; 1.4174x over previous
import functools

import jax
import jax.numpy as jnp
from jax.experimental import pallas as pl
from jax.experimental.pallas import tpu as pltpu

_NS = 8


def _dense_body(x_ref, row_ref, cs_ref):
    j = pl.program_id(1)
    xb = x_ref[0]
    G, _, W = xb.shape
    colmax = jnp.max(xb, axis=1)
    gm = jnp.max(colmax, axis=1, keepdims=True)
    ties = (xb >= gm[:, :, None]).astype(jnp.float32)
    ones = jnp.ones((G, 1, W), jnp.float32)
    rp = jax.lax.dot_general(
        ones, ties, (((2,), (2,)), ((0,), (0,))),
        preferred_element_type=jnp.float32)
    partial = jnp.sum(rp[:, 0, :], axis=0)

    @pl.when(j == 0)
    def _():
        row_ref[0, 0, :] = partial

    @pl.when(j > 0)
    def _():
        row_ref[0, 0, :] = row_ref[0, 0, :] + partial

    cs_ref[0, 0, 0, :] = jnp.sum(jnp.sum(xb, axis=1), axis=1)


def _post_body(row_ref, cs_ref, out_ref, *, C, H, W):
    row = row_ref[:, 0, :]
    B = row.shape[0]
    tri = (jax.lax.broadcasted_iota(jnp.int32, (H, H), 0)
           < jax.lax.broadcasted_iota(jnp.int32, (H, H), 1)).astype(jnp.float32)
    Hh = jax.lax.dot_general(row, tri, (((1,), (0,)), ((), ())),
                             preferred_element_type=jnp.float32)
    lane = jax.lax.broadcasted_iota(jnp.int32, (B, H), 1).astype(jnp.float32)
    jlo, jhi = 1.0, float(H - 2)
    inrange = (lane >= jlo) & (lane <= jhi)
    Hnext = jnp.concatenate([Hh[:, 1:], jnp.zeros((B, 1), jnp.float32)], axis=1)
    BIG = jnp.float32(1e9)
    hk_prev = jnp.zeros((B, 1), jnp.float32)
    valid = jnp.ones((B, 1), jnp.bool_)
    hks = [jnp.zeros((B, 1), jnp.float32)]
    for k in range(1, _NS):
        thr = float(int(k * C / _NS))
        cond = inrange & (Hh <= thr) & (Hnext > thr)
        a = jnp.min(jnp.where(cond, lane, BIG), axis=1, keepdims=True)
        b = jnp.max(jnp.where(cond, lane, -BIG), axis=1, keepdims=True)
        jk = jnp.maximum(a, hk_prev + 1.0)
        valid = valid & (jk <= b)
        hks.append(jnp.where(valid, jk, 0.0))
        hk_prev = jnp.where(valid, jk, hk_prev)
    hks.append(jnp.full((B, 1), jnp.float32(H)))
    hks = jnp.concatenate(hks, axis=1)
    widths = hks[:, 1:] - hks[:, :-1]
    F = cs_ref[:, 0, :] * jnp.float32(1.0 / W)
    out_ref[...] = F[:, None, :] / widths[:, :, None]


def _dense(x, b0, nb):
    B, C, H, W = x.shape
    G = 96
    while C % G:
        G -= 1
    nj = C // G
    row, cs = pl.pallas_call(
        _dense_body,
        grid=(nb, nj),
        in_specs=[pl.BlockSpec((1, G, H, W), lambda b, j: (b + b0, j, 0, 0))],
        out_specs=[
            pl.BlockSpec((1, 1, H), lambda b, j: (b, 0, 0)),
            pl.BlockSpec((1, 1, 1, G), lambda b, j: (b, j, 0, 0)),
        ],
        out_shape=[
            jax.ShapeDtypeStruct((nb, 1, H), jnp.float32),
            jax.ShapeDtypeStruct((nb, nj, 1, G), jnp.float32),
        ],
        compiler_params=pltpu.CompilerParams(
            dimension_semantics=("parallel", "arbitrary")),
    )(x)
    return row.reshape(nb, H), cs.reshape(nb, C)


def _abp(x):
    B, C, H, W = x.shape
    row, cs = _dense(x, 0, B)
    out = pl.pallas_call(
        functools.partial(_post_body, C=C, H=H, W=W),
        out_shape=jax.ShapeDtypeStruct((B, _NS, C), jnp.float32),
    )(row.reshape(B, 1, H), cs.reshape(B, 1, C))
    return out.reshape(B, _NS * C)


def kernel(x):
    return _abp(x)
